# baseline (speedup 1.0000x reference)
import functools
import math

import jax
import jax.numpy as jnp
from jax import lax
from jax.experimental import pallas as pl
from jax.experimental.pallas import tpu as pltpu
from jax.experimental.pallas import tpu_sc as plsc

N = 50000
E = 800000
IN = 13
H = 128
C = 10
G = 64

NC = 2
NS = 16
FCH = 4
FW = H // FCH

R = 2000
NBLK = N // R
NP = 50048
ST = NP // NS
K = 400
J = 5
NBLKE = E // (J * K)
NBPT = NBLKE // NS
ET = E // NS
ETD = E // (NC * NS)
KD = 1000
BNS = float(1.0 / math.sqrt(1.0 + 1e-5))


def _src4_body(s_ref, o_ref):
    s = s_ref[...]
    for p in range(FCH):
        o_ref[p] = s + jnp.int32(p * N)


def _make_src4(src):
    s2 = src.reshape(400, 2000)
    out = pl.pallas_call(
        _src4_body,
        grid=(50,),
        in_specs=[pl.BlockSpec((8, 2000), lambda i: (i, 0))],
        out_specs=pl.BlockSpec((FCH, 8, 2000), lambda i: (0, i, 0)),
        out_shape=jax.ShapeDtypeStruct((FCH, 400, 2000), jnp.int32),
    )(s2)
    return out.reshape(FCH * E)


def _a1_body(x_ref, w_ref, ca_ref, cb_ref, hs_ref, dinv_ref):
    cnt = ca_ref[0, :, :1] + cb_ref[0, :, :1]
    dinv = lax.rsqrt(1.0 + cnt)
    dinv_ref[...] = dinv
    y = jnp.dot(x_ref[...], w_ref[...], preferred_element_type=jnp.float32)
    y = y * dinv
    for p in range(FCH):
        hs_ref[p] = y[:, p * FW:(p + 1) * FW]


def _a1(x, w1, cnt2):
    return pl.pallas_call(
        _a1_body,
        grid=(NBLK,),
        in_specs=[
            pl.BlockSpec((R, IN), lambda i: (i, 0)),
            pl.BlockSpec((IN, H), lambda i: (0, 0)),
            pl.BlockSpec((1, R, 16), lambda i: (0, i, 0)),
            pl.BlockSpec((1, R, 16), lambda i: (1, i, 0)),
        ],
        out_specs=[
            pl.BlockSpec((FCH, R, FW), lambda i: (0, i, 0)),
            pl.BlockSpec((R, 1), lambda i: (i, 0)),
        ],
        out_shape=[
            jax.ShapeDtypeStruct((FCH, N, FW), jnp.float32),
            jax.ShapeDtypeStruct((N, 1), jnp.float32),
        ],
    )(x, w1, cnt2, cnt2)


def _a_body(h_ref, w_ref, dinv_ref, hs_ref):
    y = jnp.dot(h_ref[...], w_ref[...], preferred_element_type=jnp.float32)
    y = y * dinv_ref[...]
    for p in range(FCH):
        hs_ref[p] = y[:, p * FW:(p + 1) * FW]


def _a(h, w, dinv):
    return pl.pallas_call(
        _a_body,
        grid=(NBLK,),
        in_specs=[
            pl.BlockSpec((R, H), lambda i: (i, 0)),
            pl.BlockSpec((H, H), lambda i: (0, 0)),
            pl.BlockSpec((R, 1), lambda i: (i, 0)),
        ],
        out_specs=pl.BlockSpec((FCH, R, FW), lambda i: (0, i, 0)),
        out_shape=jax.ShapeDtypeStruct((FCH, N, FW), jnp.float32),
    )(h, w, dinv)


def _b_body(t_ref, hs_ref, dinv_ref, b_ref, g_ref, be_ref, out_ref):
    dinv = dinv_ref[...]
    for p in range(FCH):
        sl = slice(p * FW, (p + 1) * FW)
        v = dinv * (t_ref[p] + hs_ref[p]) + b_ref[:, sl]
        v = v * (g_ref[:, sl] * BNS) + be_ref[:, sl]
        out_ref[:, sl] = jnp.maximum(v, 0.0)


def _b(t4, hs4, dinv, b, g, be):
    return pl.pallas_call(
        _b_body,
        grid=(NBLK,),
        in_specs=[
            pl.BlockSpec((FCH, R, FW), lambda i: (0, i, 0)),
            pl.BlockSpec((FCH, R, FW), lambda i: (0, i, 0)),
            pl.BlockSpec((R, 1), lambda i: (i, 0)),
            pl.BlockSpec((1, H), lambda i: (0, 0)),
            pl.BlockSpec((1, H), lambda i: (0, 0)),
            pl.BlockSpec((1, H), lambda i: (0, 0)),
        ],
        out_specs=pl.BlockSpec((R, H), lambda i: (i, 0)),
        out_shape=jax.ShapeDtypeStruct((N, H), jnp.float32),
    )(t4, hs4, dinv, b, g, be)


def _b3_body(t_ref, hs_ref, dinv_ref, b_ref, g_ref, be_ref, bat_ref,
             sums_ref, cnt_ref, acc, cacc):
    i = pl.program_id(0)

    @pl.when(i == 0)
    def _():
        acc[...] = jnp.zeros_like(acc)
        cacc[...] = jnp.zeros_like(cacc)

    dinv = dinv_ref[...]
    cols = []
    for p in range(FCH):
        sl = slice(p * FW, (p + 1) * FW)
        v = dinv * (t_ref[p] + hs_ref[p]) + b_ref[:, sl]
        v = v * (g_ref[:, sl] * BNS) + be_ref[:, sl]
        cols.append(jnp.maximum(v, 0.0))
    h = jnp.concatenate(cols, axis=1)

    oh = (bat_ref[...] == lax.broadcasted_iota(jnp.int32, (R, G), 1))
    oh = oh.astype(jnp.float32)
    acc[...] += lax.dot_general(oh, h, (((0,), (0,)), ((), ())),
                                preferred_element_type=jnp.float32)
    cacc[...] += lax.dot_general(oh, jnp.ones((R, 1), jnp.float32),
                                 (((0,), (0,)), ((), ())),
                                 preferred_element_type=jnp.float32)

    @pl.when(i == NBLK - 1)
    def _():
        sums_ref[...] = acc[...]
        cnt_ref[...] = cacc[...]


def _b3(t4, hs4, dinv, b, g, be, batch):
    return pl.pallas_call(
        _b3_body,
        grid=(NBLK,),
        in_specs=[
            pl.BlockSpec((FCH, R, FW), lambda i: (0, i, 0)),
            pl.BlockSpec((FCH, R, FW), lambda i: (0, i, 0)),
            pl.BlockSpec((R, 1), lambda i: (i, 0)),
            pl.BlockSpec((1, H), lambda i: (0, 0)),
            pl.BlockSpec((1, H), lambda i: (0, 0)),
            pl.BlockSpec((1, H), lambda i: (0, 0)),
            pl.BlockSpec((R, 1), lambda i: (i, 0)),
        ],
        out_specs=[
            pl.BlockSpec((G, H), lambda i: (0, 0)),
            pl.BlockSpec((G, 1), lambda i: (0, 0)),
        ],
        out_shape=[
            jax.ShapeDtypeStruct((G, H), jnp.float32),
            jax.ShapeDtypeStruct((G, 1), jnp.float32),
        ],
        scratch_shapes=[
            pltpu.VMEM((G, H), jnp.float32),
            pltpu.VMEM((G, 1), jnp.float32),
        ],
    )(t4, hs4, dinv, b, g, be, batch)


def _c_body(s_ref, c_ref, w1_ref, b1_ref, w2_ref, b2_ref, o_ref):
    pooled = s_ref[...] / jnp.maximum(c_ref[...], 1.0)
    z = jnp.dot(pooled, w1_ref[...], preferred_element_type=jnp.float32)
    z = jnp.maximum(z + b1_ref[...], 0.0)
    o_ref[...] = jnp.dot(z, w2_ref[...],
                         preferred_element_type=jnp.float32) + b2_ref[...]


def _c(sums, cnt, fc1W, fc1b, fc2W, fc2b):
    return pl.pallas_call(
        _c_body,
        out_shape=jax.ShapeDtypeStruct((G, C), jnp.float32),
    )(sums, cnt, fc1W, fc1b, fc2W, fc2b)


def _mesh():
    return plsc.VectorSubcoreMesh(core_axis_name="c", subcore_axis_name="s",
                                  num_cores=NC, num_subcores=NS)


def _deg_kernel(ei_hbm, ones_hbm, zeros_hbm, cnt_hbm, dstb, onesb, acc):
    cid = lax.axis_index("c")
    sid = lax.axis_index("s")
    lo = sid * ST
    pltpu.sync_copy(zeros_hbm, acc.at[pl.ds(lo, ST)])
    pltpu.sync_copy(ones_hbm, onesb)
    plsc.subcore_barrier()

    base = (cid * NS + sid) * ETD

    def body(ci, _):
        off = base + ci * KD
        pltpu.sync_copy(ei_hbm.at[1, pl.ds(off, KD)], dstb)
        pltpu.sync_copy(onesb, acc.at[dstb], add=True)
        return 0

    lax.fori_loop(0, ETD // KD, body, 0, unroll=False)
    plsc.subcore_barrier()
    pltpu.sync_copy(acc.at[pl.ds(lo, ST)], cnt_hbm.at[cid, pl.ds(lo, ST)])


def _degrees(ei, ones_c, zeros16_c):
    f = functools.partial(
        pl.kernel,
        out_type=jax.ShapeDtypeStruct((NC, NP, 16), jnp.float32),
        mesh=_mesh(),
        scratch_types=[
            pltpu.VMEM((KD,), jnp.int32),
            pltpu.VMEM((KD, 16), jnp.float32),
            pltpu.VMEM_SHARED((NP, 16), jnp.float32),
        ],
        compiler_params=pltpu.CompilerParams(use_tc_tiling_on_sc=False),
    )(_deg_kernel)
    return f(ei, ones_c, zeros16_c)


def _prop_kernel(ei_hbm, hs_hbm, zeros_hbm, t_hbm,
                 idxb, d0, d1, d2, d3, d4, rows0, rows1, acc, gsem, ssem):
    cid = lax.axis_index("c")
    sid = lax.axis_index("s")
    lo = sid * ST
    rows = (rows0, rows1)
    dbufs = (d0, d1, d2, d3, d4)

    def pass_body(q, _):
        p = cid * NC + q
        pltpu.sync_copy(zeros_hbm, acc.at[pl.ds(lo, ST)])
        plsc.subcore_barrier()

        def body(b, _):
            off = sid * ET + b * (J * K)
            pltpu.sync_copy(ei_hbm.at[0, pl.ds(off, J * K)], idxb)
            for j in range(J):
                pltpu.sync_copy(ei_hbm.at[1, pl.ds(off + j * K, K)], dbufs[j])
            gd = [None, None]
            sd = [None, None]
            gd[0] = pltpu.async_copy(
                hs_hbm.at[p].at[idxb.at[pl.ds(0, K)]], rows[0], gsem)
            for j in range(J):
                pj = j % 2
                nj = (j + 1) % 2
                if j + 1 < J:
                    if sd[nj] is not None:
                        sd[nj].wait()
                    gd[nj] = pltpu.async_copy(
                        hs_hbm.at[p].at[idxb.at[pl.ds((j + 1) * K, K)]],
                        rows[nj], gsem)
                gd[pj].wait()
                sd[pj] = pltpu.async_copy(rows[pj], acc.at[dbufs[j]],
                                          ssem, add=True)
            sd[(J - 1) % 2].wait()
            if J > 1:
                sd[J % 2].wait()
            return 0

        lax.fori_loop(0, NBPT, body, 0, unroll=False)
        plsc.subcore_barrier()
        pltpu.sync_copy(acc.at[pl.ds(lo, ST)], t_hbm.at[p, pl.ds(lo, ST)])
        plsc.subcore_barrier()
        return 0

    lax.fori_loop(0, NC, pass_body, 0, unroll=False)


def _propagate(ei, hs3, zeros32_c):
    f = functools.partial(
        pl.kernel,
        out_type=jax.ShapeDtypeStruct((FCH, NP, FW), jnp.float32),
        mesh=_mesh(),
        scratch_types=[
            pltpu.VMEM((J * K,), jnp.int32),
            pltpu.VMEM((K,), jnp.int32),
            pltpu.VMEM((K,), jnp.int32),
            pltpu.VMEM((K,), jnp.int32),
            pltpu.VMEM((K,), jnp.int32),
            pltpu.VMEM((K,), jnp.int32),
            pltpu.VMEM((K, FW), jnp.float32),
            pltpu.VMEM((K, FW), jnp.float32),
            pltpu.VMEM_SHARED((NP, FW), jnp.float32),
            pltpu.SemaphoreType.DMA,
            pltpu.SemaphoreType.DMA,
        ],
        compiler_params=pltpu.CompilerParams(use_tc_tiling_on_sc=False),
    )(_prop_kernel)
    return f(ei, hs3, zeros32_c)


def kernel(x, edge_index, batch, W1, b1, W2, b2, W3, b3,
           g1, be1, g2, be2, g3, be3, fc1W, fc1b, fc2W, fc2b):
    ones_c = jnp.ones((KD, 16), jnp.float32)
    zeros16_c = jnp.zeros((ST, 16), jnp.float32)
    zeros32_c = jnp.zeros((ST, FW), jnp.float32)

    cnt2 = _degrees(edge_index, ones_c, zeros16_c)

    hs1, dinv = _a1(x, W1, cnt2)
    t1 = _propagate(edge_index, hs1, zeros32_c)
    h1 = _b(t1, hs1, dinv,
            b1.reshape(1, H), g1.reshape(1, H), be1.reshape(1, H))

    hs2 = _a(h1, W2, dinv)
    t2 = _propagate(edge_index, hs2, zeros32_c)
    h2 = _b(t2, hs2, dinv,
            b2.reshape(1, H), g2.reshape(1, H), be2.reshape(1, H))

    hs3 = _a(h2, W3, dinv)
    t3 = _propagate(edge_index, hs3, zeros32_c)
    sums, cnt = _b3(t3, hs3, dinv,
                    b3.reshape(1, H), g3.reshape(1, H), be3.reshape(1, H),
                    batch.reshape(N, 1))

    return _c(sums, cnt, fc1W, fc1b.reshape(1, H // 2),
              fc2W, fc2b.reshape(1, C))


# speedup vs baseline: 1.4641x; 1.4641x over previous
import functools
import math

import jax
import jax.numpy as jnp
from jax import lax
from jax.experimental import pallas as pl
from jax.experimental.pallas import tpu as pltpu
from jax.experimental.pallas import tpu_sc as plsc

N = 50000
E = 800000
IN = 13
H = 128
C = 10
G = 64

NC = 2
NS = 16
FCH = 4
FW = H // FCH

R = 2000
NBLK = N // R
NP = 50048
ST = NP // NS
K = 400
J = 5
NBPT = E // (J * K * NS)
ET = E // NS
ETD = E // (NC * NS)
KD = 1000
BNS = float(1.0 / math.sqrt(1.0 + 1e-5))


def _a1_body(x_ref, w_ref, ca_ref, cb_ref, hs_ref, dinv_ref):
    cnt = ca_ref[0, :, :1] + cb_ref[0, :, :1]
    dinv = lax.rsqrt(1.0 + cnt)
    dinv_ref[...] = dinv
    y = jnp.dot(x_ref[...], w_ref[...], preferred_element_type=jnp.float32)
    hs_ref[...] = y * dinv


def _a1(x, w1, cnt2):
    return pl.pallas_call(
        _a1_body,
        grid=(NBLK,),
        in_specs=[
            pl.BlockSpec((R, IN), lambda i: (i, 0)),
            pl.BlockSpec((IN, H), lambda i: (0, 0)),
            pl.BlockSpec((1, R, 16), lambda i: (0, i, 0)),
            pl.BlockSpec((1, R, 16), lambda i: (1, i, 0)),
        ],
        out_specs=[
            pl.BlockSpec((R, H), lambda i: (i, 0)),
            pl.BlockSpec((R, 1), lambda i: (i, 0)),
        ],
        out_shape=[
            jax.ShapeDtypeStruct((N, H), jnp.float32),
            jax.ShapeDtypeStruct((N, 1), jnp.float32),
        ],
    )(x, w1, cnt2, cnt2)


def _a_body(h_ref, w_ref, dinv_ref, hs_ref):
    y = jnp.dot(h_ref[...], w_ref[...], preferred_element_type=jnp.float32)
    hs_ref[...] = y * dinv_ref[...]


def _a(h, w, dinv):
    return pl.pallas_call(
        _a_body,
        grid=(NBLK,),
        in_specs=[
            pl.BlockSpec((R, H), lambda i: (i, 0)),
            pl.BlockSpec((H, H), lambda i: (0, 0)),
            pl.BlockSpec((R, 1), lambda i: (i, 0)),
        ],
        out_specs=pl.BlockSpec((R, H), lambda i: (i, 0)),
        out_shape=jax.ShapeDtypeStruct((N, H), jnp.float32),
    )(h, w, dinv)


def _b_body(t_ref, hs_ref, dinv_ref, b_ref, g_ref, be_ref, out_ref):
    v = dinv_ref[...] * (t_ref[...] + hs_ref[...]) + b_ref[...]
    v = v * (g_ref[...] * BNS) + be_ref[...]
    out_ref[...] = jnp.maximum(v, 0.0)


def _b(t, hs, dinv, b, g, be):
    return pl.pallas_call(
        _b_body,
        grid=(NBLK,),
        in_specs=[
            pl.BlockSpec((R, H), lambda i: (i, 0)),
            pl.BlockSpec((R, H), lambda i: (i, 0)),
            pl.BlockSpec((R, 1), lambda i: (i, 0)),
            pl.BlockSpec((1, H), lambda i: (0, 0)),
            pl.BlockSpec((1, H), lambda i: (0, 0)),
            pl.BlockSpec((1, H), lambda i: (0, 0)),
        ],
        out_specs=pl.BlockSpec((R, H), lambda i: (i, 0)),
        out_shape=jax.ShapeDtypeStruct((N, H), jnp.float32),
    )(t, hs, dinv, b, g, be)


def _b3_body(t_ref, hs_ref, dinv_ref, b_ref, g_ref, be_ref, bat_ref,
             sums_ref, cnt_ref, acc, cacc):
    i = pl.program_id(0)

    @pl.when(i == 0)
    def _():
        acc[...] = jnp.zeros_like(acc)
        cacc[...] = jnp.zeros_like(cacc)

    v = dinv_ref[...] * (t_ref[...] + hs_ref[...]) + b_ref[...]
    v = v * (g_ref[...] * BNS) + be_ref[...]
    h = jnp.maximum(v, 0.0)

    oh = (bat_ref[...] == lax.broadcasted_iota(jnp.int32, (R, G), 1))
    oh = oh.astype(jnp.float32)
    acc[...] += lax.dot_general(oh, h, (((0,), (0,)), ((), ())),
                                preferred_element_type=jnp.float32)
    cacc[...] += lax.dot_general(oh, jnp.ones((R, 1), jnp.float32),
                                 (((0,), (0,)), ((), ())),
                                 preferred_element_type=jnp.float32)

    @pl.when(i == NBLK - 1)
    def _():
        sums_ref[...] = acc[...]
        cnt_ref[...] = cacc[...]


def _b3(t, hs, dinv, b, g, be, batch):
    return pl.pallas_call(
        _b3_body,
        grid=(NBLK,),
        in_specs=[
            pl.BlockSpec((R, H), lambda i: (i, 0)),
            pl.BlockSpec((R, H), lambda i: (i, 0)),
            pl.BlockSpec((R, 1), lambda i: (i, 0)),
            pl.BlockSpec((1, H), lambda i: (0, 0)),
            pl.BlockSpec((1, H), lambda i: (0, 0)),
            pl.BlockSpec((1, H), lambda i: (0, 0)),
            pl.BlockSpec((R, 1), lambda i: (i, 0)),
        ],
        out_specs=[
            pl.BlockSpec((G, H), lambda i: (0, 0)),
            pl.BlockSpec((G, 1), lambda i: (0, 0)),
        ],
        out_shape=[
            jax.ShapeDtypeStruct((G, H), jnp.float32),
            jax.ShapeDtypeStruct((G, 1), jnp.float32),
        ],
        scratch_shapes=[
            pltpu.VMEM((G, H), jnp.float32),
            pltpu.VMEM((G, 1), jnp.float32),
        ],
    )(t, hs, dinv, b, g, be, batch)


def _c_body(s_ref, c_ref, w1_ref, b1_ref, w2_ref, b2_ref, o_ref):
    pooled = s_ref[...] / jnp.maximum(c_ref[...], 1.0)
    z = jnp.dot(pooled, w1_ref[...], preferred_element_type=jnp.float32)
    z = jnp.maximum(z + b1_ref[...], 0.0)
    o_ref[...] = jnp.dot(z, w2_ref[...],
                         preferred_element_type=jnp.float32) + b2_ref[...]


def _c(sums, cnt, fc1W, fc1b, fc2W, fc2b):
    return pl.pallas_call(
        _c_body,
        out_shape=jax.ShapeDtypeStruct((G, C), jnp.float32),
    )(sums, cnt, fc1W, fc1b, fc2W, fc2b)


def _mesh():
    return plsc.VectorSubcoreMesh(core_axis_name="c", subcore_axis_name="s",
                                  num_cores=NC, num_subcores=NS)


def _deg_kernel(ei_hbm, ones_hbm, zeros_hbm, cnt_hbm, dstb, onesb, acc):
    cid = lax.axis_index("c")
    sid = lax.axis_index("s")
    lo = sid * ST
    pltpu.sync_copy(zeros_hbm, acc.at[pl.ds(lo, ST)])
    pltpu.sync_copy(ones_hbm, onesb)
    plsc.subcore_barrier()

    base = (cid * NS + sid) * ETD

    def body(ci, _):
        off = base + ci * KD
        pltpu.sync_copy(ei_hbm.at[1, pl.ds(off, KD)], dstb)
        pltpu.sync_copy(onesb, acc.at[dstb], add=True)
        return 0

    lax.fori_loop(0, ETD // KD, body, 0, unroll=False)
    plsc.subcore_barrier()
    pltpu.sync_copy(acc.at[pl.ds(lo, ST)], cnt_hbm.at[cid, pl.ds(lo, ST)])


def _degrees(ei, ones_c, zeros16_c):
    f = functools.partial(
        pl.kernel,
        out_type=jax.ShapeDtypeStruct((NC, NP, 16), jnp.float32),
        mesh=_mesh(),
        scratch_types=[
            pltpu.VMEM((KD,), jnp.int32),
            pltpu.VMEM((KD, 16), jnp.float32),
            pltpu.VMEM_SHARED((NP, 16), jnp.float32),
        ],
        compiler_params=pltpu.CompilerParams(use_tc_tiling_on_sc=False),
    )(_deg_kernel)
    return f(ei, ones_c, zeros16_c)


def _prop_kernel(ei_hbm, hs_hbm, zeros_hbm, t_hbm,
                 idxb, dstb, rows0, rows1, acc, gsem, ssem):
    cid = lax.axis_index("c")
    sid = lax.axis_index("s")
    lo = sid * ST
    rows = (rows0, rows1)

    def pass_body(q, _):
        p = cid * NC + q
        pltpu.sync_copy(zeros_hbm, acc.at[pl.ds(lo, ST)])
        plsc.subcore_barrier()

        def body(b, _):
            off = sid * ET + b * (J * K)
            pltpu.sync_copy(ei_hbm.at[0, pl.ds(off, J * K)], idxb)
            pltpu.sync_copy(ei_hbm.at[1, pl.ds(off, J * K)], dstb)

            def fix(i, _):
                v = idxb[pl.ds(i * 16, 16)]
                idxb[pl.ds(i * 16, 16)] = v * 4 + p
                return 0

            lax.fori_loop(0, (J * K) // 16, fix, 0, unroll=False)

            gd = [None, None]
            sd = [None, None]
            gd[0] = pltpu.async_copy(
                hs_hbm.at[idxb.at[pl.ds(0, K)]], rows[0], gsem)
            for j in range(J):
                pj = j % 2
                nj = (j + 1) % 2
                if j + 1 < J:
                    if sd[nj] is not None:
                        sd[nj].wait()
                    gd[nj] = pltpu.async_copy(
                        hs_hbm.at[idxb.at[pl.ds((j + 1) * K, K)]],
                        rows[nj], gsem)
                gd[pj].wait()
                sd[pj] = pltpu.async_copy(
                    rows[pj], acc.at[dstb.at[pl.ds(j * K, K)]],
                    ssem, add=True)
            sd[(J - 1) % 2].wait()
            if J > 1:
                sd[J % 2].wait()
            return 0

        lax.fori_loop(0, NBPT, body, 0, unroll=False)
        plsc.subcore_barrier()
        pltpu.sync_copy(acc.at[pl.ds(lo, ST)],
                        t_hbm.at[pl.ds(lo, ST), pl.ds(p * FW, FW)])
        plsc.subcore_barrier()
        return 0

    lax.fori_loop(0, NC, pass_body, 0, unroll=False)


def _propagate(ei, hs, zeros32_c):
    f = functools.partial(
        pl.kernel,
        out_type=jax.ShapeDtypeStruct((NP, H), jnp.float32),
        mesh=_mesh(),
        scratch_types=[
            pltpu.VMEM((J * K,), jnp.int32),
            pltpu.VMEM((J * K,), jnp.int32),
            pltpu.VMEM((K, FW), jnp.float32),
            pltpu.VMEM((K, FW), jnp.float32),
            pltpu.VMEM_SHARED((NP, FW), jnp.float32),
            pltpu.SemaphoreType.DMA,
            pltpu.SemaphoreType.DMA,
        ],
        compiler_params=pltpu.CompilerParams(use_tc_tiling_on_sc=False),
    )(_prop_kernel)
    return f(ei, hs.reshape(FCH * N, FW), zeros32_c)


def kernel(x, edge_index, batch, W1, b1, W2, b2, W3, b3,
           g1, be1, g2, be2, g3, be3, fc1W, fc1b, fc2W, fc2b):
    ones_c = jnp.ones((KD, 16), jnp.float32)
    zeros16_c = jnp.zeros((ST, 16), jnp.float32)
    zeros32_c = jnp.zeros((ST, FW), jnp.float32)

    cnt2 = _degrees(edge_index, ones_c, zeros16_c)

    hs1, dinv = _a1(x, W1, cnt2)
    t1 = _propagate(edge_index, hs1, zeros32_c)
    h1 = _b(t1, hs1, dinv,
            b1.reshape(1, H), g1.reshape(1, H), be1.reshape(1, H))

    hs2 = _a(h1, W2, dinv)
    t2 = _propagate(edge_index, hs2, zeros32_c)
    h2 = _b(t2, hs2, dinv,
            b2.reshape(1, H), g2.reshape(1, H), be2.reshape(1, H))

    hs3 = _a(h2, W3, dinv)
    t3 = _propagate(edge_index, hs3, zeros32_c)
    sums, cnt = _b3(t3, hs3, dinv,
                    b3.reshape(1, H), g3.reshape(1, H), be3.reshape(1, H),
                    batch.reshape(N, 1))

    return _c(sums, cnt, fc1W, fc1b.reshape(1, H // 2),
              fc2W, fc2b.reshape(1, C))


# speedup vs baseline: 1.5189x; 1.0375x over previous
import functools
import math

import jax
import jax.numpy as jnp
from jax import lax
from jax.experimental import pallas as pl
from jax.experimental.pallas import tpu as pltpu
from jax.experimental.pallas import tpu_sc as plsc

N = 50000
E = 800000
IN = 13
H = 128
C = 10
G = 64

NC = 2
NS = 16
FCH = 4
FW = H // FCH

R = 2000
NBLK = N // R
NP = 50048
ST = NP // NS
K = 400
J = 5
NBPT = E // (J * K * NS)
ET = E // NS
ETD = E // (NC * NS)
KD = 1000
BNS = float(1.0 / math.sqrt(1.0 + 1e-5))


def _src4_body(s_ref, o_ref):
    v = s_ref[...]
    for p in range(FCH):
        o_ref[p] = v * 4 + p


def _make_src4(src):
    return pl.pallas_call(
        _src4_body,
        out_shape=jax.ShapeDtypeStruct((FCH, E // 128, 128), jnp.int32),
    )(src.reshape(E // 128, 128)).reshape(FCH * E)


def _a1_body(x_ref, w_ref, ca_ref, cb_ref, hs_ref, dinv_ref):
    cnt = ca_ref[0, :, :1] + cb_ref[0, :, :1]
    dinv = lax.rsqrt(1.0 + cnt)
    dinv_ref[...] = dinv
    y = jnp.dot(x_ref[...], w_ref[...], preferred_element_type=jnp.float32)
    hs_ref[...] = y * dinv


def _a1(x, w1, cnt2):
    return pl.pallas_call(
        _a1_body,
        grid=(NBLK,),
        in_specs=[
            pl.BlockSpec((R, IN), lambda i: (i, 0)),
            pl.BlockSpec((IN, H), lambda i: (0, 0)),
            pl.BlockSpec((1, R, 16), lambda i: (0, i, 0)),
            pl.BlockSpec((1, R, 16), lambda i: (1, i, 0)),
        ],
        out_specs=[
            pl.BlockSpec((R, H), lambda i: (i, 0)),
            pl.BlockSpec((R, 1), lambda i: (i, 0)),
        ],
        out_shape=[
            jax.ShapeDtypeStruct((N, H), jnp.float32),
            jax.ShapeDtypeStruct((N, 1), jnp.float32),
        ],
    )(x, w1, cnt2, cnt2)


def _a_body(h_ref, w_ref, dinv_ref, hs_ref):
    y = jnp.dot(h_ref[...], w_ref[...], preferred_element_type=jnp.float32)
    hs_ref[...] = y * dinv_ref[...]


def _a(h, w, dinv):
    return pl.pallas_call(
        _a_body,
        grid=(NBLK,),
        in_specs=[
            pl.BlockSpec((R, H), lambda i: (i, 0)),
            pl.BlockSpec((H, H), lambda i: (0, 0)),
            pl.BlockSpec((R, 1), lambda i: (i, 0)),
        ],
        out_specs=pl.BlockSpec((R, H), lambda i: (i, 0)),
        out_shape=jax.ShapeDtypeStruct((N, H), jnp.float32),
    )(h, w, dinv)


def _b_body(t_ref, hs_ref, dinv_ref, b_ref, g_ref, be_ref, out_ref):
    v = dinv_ref[...] * (t_ref[...] + hs_ref[...]) + b_ref[...]
    v = v * (g_ref[...] * BNS) + be_ref[...]
    out_ref[...] = jnp.maximum(v, 0.0)


def _b(t, hs, dinv, b, g, be):
    return pl.pallas_call(
        _b_body,
        grid=(NBLK,),
        in_specs=[
            pl.BlockSpec((R, H), lambda i: (i, 0)),
            pl.BlockSpec((R, H), lambda i: (i, 0)),
            pl.BlockSpec((R, 1), lambda i: (i, 0)),
            pl.BlockSpec((1, H), lambda i: (0, 0)),
            pl.BlockSpec((1, H), lambda i: (0, 0)),
            pl.BlockSpec((1, H), lambda i: (0, 0)),
        ],
        out_specs=pl.BlockSpec((R, H), lambda i: (i, 0)),
        out_shape=jax.ShapeDtypeStruct((N, H), jnp.float32),
    )(t, hs, dinv, b, g, be)


def _b3_body(t_ref, hs_ref, dinv_ref, b_ref, g_ref, be_ref, bat_ref,
             sums_ref, cnt_ref, acc, cacc):
    i = pl.program_id(0)

    @pl.when(i == 0)
    def _():
        acc[...] = jnp.zeros_like(acc)
        cacc[...] = jnp.zeros_like(cacc)

    v = dinv_ref[...] * (t_ref[...] + hs_ref[...]) + b_ref[...]
    v = v * (g_ref[...] * BNS) + be_ref[...]
    h = jnp.maximum(v, 0.0)

    oh = (bat_ref[...] == lax.broadcasted_iota(jnp.int32, (R, G), 1))
    oh = oh.astype(jnp.float32)
    acc[...] += lax.dot_general(oh, h, (((0,), (0,)), ((), ())),
                                preferred_element_type=jnp.float32)
    cacc[...] += lax.dot_general(oh, jnp.ones((R, 1), jnp.float32),
                                 (((0,), (0,)), ((), ())),
                                 preferred_element_type=jnp.float32)

    @pl.when(i == NBLK - 1)
    def _():
        sums_ref[...] = acc[...]
        cnt_ref[...] = cacc[...]


def _b3(t, hs, dinv, b, g, be, batch):
    return pl.pallas_call(
        _b3_body,
        grid=(NBLK,),
        in_specs=[
            pl.BlockSpec((R, H), lambda i: (i, 0)),
            pl.BlockSpec((R, H), lambda i: (i, 0)),
            pl.BlockSpec((R, 1), lambda i: (i, 0)),
            pl.BlockSpec((1, H), lambda i: (0, 0)),
            pl.BlockSpec((1, H), lambda i: (0, 0)),
            pl.BlockSpec((1, H), lambda i: (0, 0)),
            pl.BlockSpec((R, 1), lambda i: (i, 0)),
        ],
        out_specs=[
            pl.BlockSpec((G, H), lambda i: (0, 0)),
            pl.BlockSpec((G, 1), lambda i: (0, 0)),
        ],
        out_shape=[
            jax.ShapeDtypeStruct((G, H), jnp.float32),
            jax.ShapeDtypeStruct((G, 1), jnp.float32),
        ],
        scratch_shapes=[
            pltpu.VMEM((G, H), jnp.float32),
            pltpu.VMEM((G, 1), jnp.float32),
        ],
    )(t, hs, dinv, b, g, be, batch)


def _c_body(s_ref, c_ref, w1_ref, b1_ref, w2_ref, b2_ref, o_ref):
    pooled = s_ref[...] / jnp.maximum(c_ref[...], 1.0)
    z = jnp.dot(pooled, w1_ref[...], preferred_element_type=jnp.float32)
    z = jnp.maximum(z + b1_ref[...], 0.0)
    o_ref[...] = jnp.dot(z, w2_ref[...],
                         preferred_element_type=jnp.float32) + b2_ref[...]


def _c(sums, cnt, fc1W, fc1b, fc2W, fc2b):
    return pl.pallas_call(
        _c_body,
        out_shape=jax.ShapeDtypeStruct((G, C), jnp.float32),
    )(sums, cnt, fc1W, fc1b, fc2W, fc2b)


def _mesh():
    return plsc.VectorSubcoreMesh(core_axis_name="c", subcore_axis_name="s",
                                  num_cores=NC, num_subcores=NS)


def _deg_kernel(ei_hbm, ones_hbm, zeros_hbm, cnt_hbm, dstb, onesb, acc):
    cid = lax.axis_index("c")
    sid = lax.axis_index("s")
    lo = sid * ST
    pltpu.sync_copy(zeros_hbm, acc.at[pl.ds(lo, ST)])
    pltpu.sync_copy(ones_hbm, onesb)
    plsc.subcore_barrier()

    base = (cid * NS + sid) * ETD

    def body(ci, _):
        off = base + ci * KD
        pltpu.sync_copy(ei_hbm.at[1, pl.ds(off, KD)], dstb)
        pltpu.sync_copy(onesb, acc.at[dstb], add=True)
        return 0

    lax.fori_loop(0, ETD // KD, body, 0, unroll=False)
    plsc.subcore_barrier()
    pltpu.sync_copy(acc.at[pl.ds(lo, ST)], cnt_hbm.at[cid, pl.ds(lo, ST)])


def _degrees(ei, ones_c, zeros16_c):
    f = functools.partial(
        pl.kernel,
        out_type=jax.ShapeDtypeStruct((NC, NP, 16), jnp.float32),
        mesh=_mesh(),
        scratch_types=[
            pltpu.VMEM((KD,), jnp.int32),
            pltpu.VMEM((KD, 16), jnp.float32),
            pltpu.VMEM_SHARED((NP, 16), jnp.float32),
        ],
        compiler_params=pltpu.CompilerParams(use_tc_tiling_on_sc=False),
    )(_deg_kernel)
    return f(ei, ones_c, zeros16_c)


def _prop_kernel(src4_hbm, ei_hbm, hs_hbm, zeros_hbm, t_hbm,
                 idxb, dstb, rows0, rows1, acc, gsem, ssem):
    cid = lax.axis_index("c")
    sid = lax.axis_index("s")
    lo = sid * ST
    rows = (rows0, rows1)

    def pass_body(q, _):
        p = cid * NC + q
        pltpu.sync_copy(zeros_hbm, acc.at[pl.ds(lo, ST)])
        plsc.subcore_barrier()

        def body(b, _):
            off = sid * ET + b * (J * K)
            pltpu.sync_copy(src4_hbm.at[pl.ds(p * E + off, J * K)], idxb)
            pltpu.sync_copy(ei_hbm.at[1, pl.ds(off, J * K)], dstb)

            gd = [None, None]
            sd = [None, None]
            gd[0] = pltpu.async_copy(
                hs_hbm.at[idxb.at[pl.ds(0, K)]], rows[0], gsem)
            for j in range(J):
                pj = j % 2
                nj = (j + 1) % 2
                if j + 1 < J:
                    if sd[nj] is not None:
                        sd[nj].wait()
                    gd[nj] = pltpu.async_copy(
                        hs_hbm.at[idxb.at[pl.ds((j + 1) * K, K)]],
                        rows[nj], gsem)
                gd[pj].wait()
                sd[pj] = pltpu.async_copy(
                    rows[pj], acc.at[dstb.at[pl.ds(j * K, K)]],
                    ssem, add=True)
            sd[(J - 1) % 2].wait()
            if J > 1:
                sd[J % 2].wait()
            return 0

        lax.fori_loop(0, NBPT, body, 0, unroll=False)
        plsc.subcore_barrier()
        pltpu.sync_copy(acc.at[pl.ds(lo, ST)],
                        t_hbm.at[pl.ds(lo, ST), pl.ds(p * FW, FW)])
        plsc.subcore_barrier()
        return 0

    lax.fori_loop(0, NC, pass_body, 0, unroll=False)


def _propagate(src4, ei, hs, zeros32_c):
    f = functools.partial(
        pl.kernel,
        out_type=jax.ShapeDtypeStruct((NP, H), jnp.float32),
        mesh=_mesh(),
        scratch_types=[
            pltpu.VMEM((J * K,), jnp.int32),
            pltpu.VMEM((J * K,), jnp.int32),
            pltpu.VMEM((K, FW), jnp.float32),
            pltpu.VMEM((K, FW), jnp.float32),
            pltpu.VMEM_SHARED((NP, FW), jnp.float32),
            pltpu.SemaphoreType.DMA,
            pltpu.SemaphoreType.DMA,
        ],
        compiler_params=pltpu.CompilerParams(use_tc_tiling_on_sc=False),
    )(_prop_kernel)
    return f(src4, ei, hs.reshape(FCH * N, FW), zeros32_c)


def kernel(x, edge_index, batch, W1, b1, W2, b2, W3, b3,
           g1, be1, g2, be2, g3, be3, fc1W, fc1b, fc2W, fc2b):
    ones_c = jnp.ones((KD, 16), jnp.float32)
    zeros16_c = jnp.zeros((ST, 16), jnp.float32)
    zeros32_c = jnp.zeros((ST, FW), jnp.float32)

    src4 = _make_src4(edge_index[0])
    cnt2 = _degrees(edge_index, ones_c, zeros16_c)

    hs1, dinv = _a1(x, W1, cnt2)
    t1 = _propagate(src4, edge_index, hs1, zeros32_c)
    h1 = _b(t1, hs1, dinv,
            b1.reshape(1, H), g1.reshape(1, H), be1.reshape(1, H))

    hs2 = _a(h1, W2, dinv)
    t2 = _propagate(src4, edge_index, hs2, zeros32_c)
    h2 = _b(t2, hs2, dinv,
            b2.reshape(1, H), g2.reshape(1, H), be2.reshape(1, H))

    hs3 = _a(h2, W3, dinv)
    t3 = _propagate(src4, edge_index, hs3, zeros32_c)
    sums, cnt = _b3(t3, hs3, dinv,
                    b3.reshape(1, H), g3.reshape(1, H), be3.reshape(1, H),
                    batch.reshape(N, 1))

    return _c(sums, cnt, fc1W, fc1b.reshape(1, H // 2),
              fc2W, fc2b.reshape(1, C))
